# baseline (speedup 1.0000x reference)
import functools

import jax
import jax.numpy as jnp
from jax import lax
from jax.experimental import pallas as pl
from jax.experimental.pallas import tpu as pltpu
from jax.experimental.pallas import tpu_sc as plsc

_B, _S, _D = 4, 2048, 1024
_N = _B * _S
_NC, _NS = 2, 16
_NW = _NC * _NS
_PER_W = _N // _NW
_CHUNK = 8
_NCHUNK = _PER_W // _CHUNK
_NBUF = 12
_WLAG = 5

_mesh = plsc.VectorSubcoreMesh(core_axis_name="c", subcore_axis_name="s")


@functools.partial(
    pl.kernel,
    mesh=_mesh,
    out_type=jax.ShapeDtypeStruct((_B, _S, _D), jnp.float32),
    scratch_types=(
        [pltpu.VMEM((_PER_W,), jnp.int32)]
        + [pltpu.VMEM((_CHUNK, _D), jnp.float32) for _ in range(_NBUF)]
        + [pltpu.SemaphoreType.DMA for _ in range(2 * _NBUF)]
    ),
)
def _emb_gather(idx_hbm, table_hbm, out_hbm, idx_v, *bufs_and_sems):
    bufs = bufs_and_sems[:_NBUF]
    gsems = bufs_and_sems[_NBUF : 2 * _NBUF]
    wsems = bufs_and_sems[2 * _NBUF :]

    wid = lax.axis_index("s") * _NC + lax.axis_index("c")
    brow = wid // (_S // _PER_W)
    col0 = (wid % (_S // _PER_W)) * _PER_W

    pltpu.sync_copy(idx_hbm.at[brow, pl.ds(col0, _PER_W)], idx_v)

    def gather(j):
        b = j % _NBUF
        return pltpu.async_copy(
            table_hbm.at[idx_v.at[pl.ds(j * _CHUNK, _CHUNK)]], bufs[b], gsems[b]
        )

    def write(j):
        b = j % _NBUF
        return pltpu.async_copy(
            bufs[b], out_hbm.at[brow, pl.ds(col0 + j * _CHUNK, _CHUNK)], wsems[b]
        )

    gh = {}
    wh = {}
    for j in range(min(_NBUF, _NCHUNK)):
        gh[j] = gather(j)
    for j in range(_NCHUNK):
        gh[j].wait()
        wh[j] = write(j)
        k = j - _WLAG
        if k >= 0 and k + _NBUF < _NCHUNK:
            wh[k].wait()
            gh[k + _NBUF] = gather(k + _NBUF)
    for j in range(max(0, _NCHUNK - _NBUF), _NCHUNK):
        wh[j].wait()


def kernel(inputs, weight):
    return _emb_gather(inputs.astype(jnp.int32), weight)


# speedup vs baseline: 1.0137x; 1.0137x over previous
import functools

import jax
import jax.numpy as jnp
from jax import lax
from jax.experimental import pallas as pl
from jax.experimental.pallas import tpu as pltpu
from jax.experimental.pallas import tpu_sc as plsc

_B, _S, _D = 4, 2048, 1024
_N = _B * _S
_NC, _NS = 2, 16
_NW = _NC * _NS
_PER_W = _N // _NW
_CHUNK = 16
_NCHUNK = _PER_W // _CHUNK
_NBUF = 7
_WLAG = 2

_mesh = plsc.VectorSubcoreMesh(core_axis_name="c", subcore_axis_name="s")


@functools.partial(
    pl.kernel,
    mesh=_mesh,
    out_type=jax.ShapeDtypeStruct((_B, _S, _D), jnp.float32),
    scratch_types=(
        [pltpu.VMEM((_PER_W,), jnp.int32)]
        + [pltpu.VMEM((_CHUNK, _D), jnp.float32) for _ in range(_NBUF)]
        + [pltpu.SemaphoreType.DMA for _ in range(2 * _NBUF)]
    ),
)
def _emb_gather(idx_hbm, table_hbm, out_hbm, idx_v, *bufs_and_sems):
    bufs = bufs_and_sems[:_NBUF]
    gsems = bufs_and_sems[_NBUF : 2 * _NBUF]
    wsems = bufs_and_sems[2 * _NBUF :]

    wid = lax.axis_index("s") * _NC + lax.axis_index("c")
    brow = wid // (_S // _PER_W)
    col0 = (wid % (_S // _PER_W)) * _PER_W

    pltpu.sync_copy(idx_hbm.at[brow, pl.ds(col0, _PER_W)], idx_v)

    def gather(j):
        b = j % _NBUF
        return pltpu.async_copy(
            table_hbm.at[idx_v.at[pl.ds(j * _CHUNK, _CHUNK)]], bufs[b], gsems[b]
        )

    def write(j):
        b = j % _NBUF
        return pltpu.async_copy(
            bufs[b], out_hbm.at[brow, pl.ds(col0 + j * _CHUNK, _CHUNK)], wsems[b]
        )

    gh = {}
    wh = {}
    for j in range(min(_NBUF, _NCHUNK)):
        gh[j] = gather(j)
    for j in range(_NCHUNK):
        gh[j].wait()
        wh[j] = write(j)
        k = j - _WLAG
        if k >= 0 and k + _NBUF < _NCHUNK:
            wh[k].wait()
            gh[k + _NBUF] = gather(k + _NBUF)
    for j in range(max(0, _NCHUNK - _NBUF), _NCHUNK):
        wh[j].wait()


def kernel(inputs, weight):
    return _emb_gather(inputs.astype(jnp.int32), weight)
